# baseline (device time: 25578 ns/iter reference)
import jax
import jax.numpy as jnp
from jax import lax
from jax.experimental import pallas as pl
from jax.experimental.pallas import tpu as pltpu

_DIMS = (((1,), (0,)), ((), ()))
_C = 8


def kernel(x, dy):
    m, d = x.shape
    _, f = dy.shape
    half = d // 2
    zh = half // 2
    fc = f // _C

    def body(x_ref, dy_ref, out_ref, dyv, ysend, yrecv, zsend, zrecv, xt,
             dy_sem, out_sem, ysend_sem, yrecv_sem, zsend_sem, zrecv_sem):
        my_x = lax.axis_index("x")
        my_y = lax.axis_index("y")
        my_z = lax.axis_index("z")
        ypartner = (my_x, 1 - my_y, my_z)
        zpartner = (my_x, my_y, 1 - my_z)

        dy_cps = []
        for i in range(_C):
            cp = pltpu.make_async_copy(
                dy_ref.at[:, pl.ds(i * fc, fc)], dyv.at[i], dy_sem.at[i]
            )
            cp.start()
            dy_cps.append(cp)

        barrier_sem = pltpu.get_barrier_semaphore()
        for nbr in (ypartner, zpartner):
            pl.semaphore_signal(
                barrier_sem, inc=1, device_id=nbr,
                device_id_type=pl.DeviceIdType.MESH,
            )
        pl.semaphore_wait(barrier_sem, 2)

        xt[0] = x_ref[:, pl.ds((1 - my_y) * half + my_z * zh, zh)].T
        xt[1] = x_ref[:, pl.ds(my_y * half + my_z * zh, zh)].T
        xs = xt[0]
        xo = xt[1]

        y_rdmas = []
        for i in range(_C):
            dy_cps[i].wait()
            ysend[i] = lax.dot_general(
                xs, dyv[i], _DIMS, preferred_element_type=jnp.float32
            )
            r = pltpu.make_async_remote_copy(
                src_ref=ysend.at[i], dst_ref=yrecv.at[i],
                send_sem=ysend_sem.at[i], recv_sem=yrecv_sem.at[i],
                device_id=ypartner, device_id_type=pl.DeviceIdType.MESH,
            )
            r.start()
            y_rdmas.append(r)

        z_rdmas = []
        out_cps = []
        for i in range(_C):
            own = lax.dot_general(
                xo, dyv[i], _DIMS, preferred_element_type=jnp.float32
            )
            y_rdmas[i].wait_recv()
            red = own + yrecv[i]
            zsend[i] = red
            r = pltpu.make_async_remote_copy(
                src_ref=zsend.at[i], dst_ref=zrecv.at[i],
                send_sem=zsend_sem.at[i], recv_sem=zrecv_sem.at[i],
                device_id=zpartner, device_id_type=pl.DeviceIdType.MESH,
            )
            r.start()
            z_rdmas.append(r)
            cp = pltpu.make_async_copy(
                zsend.at[i],
                out_ref.at[pl.ds(my_z * zh, zh), pl.ds(i * fc, fc)],
                out_sem.at[i],
            )
            cp.start()
            out_cps.append(cp)

        for i in range(_C):
            z_rdmas[i].wait_recv()
            cp = pltpu.make_async_copy(
                zrecv.at[i],
                out_ref.at[pl.ds((1 - my_z) * zh, zh), pl.ds(i * fc, fc)],
                out_sem.at[_C + i],
            )
            cp.start()
            out_cps.append(cp)

        for cp in out_cps:
            cp.wait()
        for i in range(_C):
            y_rdmas[i].wait_send()
            z_rdmas[i].wait_send()

    return pl.pallas_call(
        body,
        out_shape=jax.ShapeDtypeStruct((half, f), jnp.float32),
        in_specs=[
            pl.BlockSpec(memory_space=pltpu.VMEM),
            pl.BlockSpec(memory_space=pltpu.MemorySpace.HBM),
        ],
        out_specs=pl.BlockSpec(memory_space=pltpu.MemorySpace.HBM),
        scratch_shapes=[
            pltpu.VMEM((_C, m, fc), jnp.float32),
            pltpu.VMEM((_C, zh, fc), jnp.float32),
            pltpu.VMEM((_C, zh, fc), jnp.float32),
            pltpu.VMEM((_C, zh, fc), jnp.float32),
            pltpu.VMEM((_C, zh, fc), jnp.float32),
            pltpu.VMEM((2, zh, m), jnp.float32),
            pltpu.SemaphoreType.DMA((_C,)),
            pltpu.SemaphoreType.DMA((2 * _C,)),
            pltpu.SemaphoreType.DMA((_C,)),
            pltpu.SemaphoreType.DMA((_C,)),
            pltpu.SemaphoreType.DMA((_C,)),
            pltpu.SemaphoreType.DMA((_C,)),
        ],
        compiler_params=pltpu.CompilerParams(collective_id=0),
    )(x, dy)


# device time: 18798 ns/iter; 1.3607x vs baseline; 1.3607x over previous
import jax
import jax.numpy as jnp
from jax import lax
from jax.experimental import pallas as pl
from jax.experimental.pallas import tpu as pltpu

_DIMS = (((1,), (0,)), ((), ()))
_C = 8


def kernel(x, dy):
    m, d = x.shape
    _, f = dy.shape
    half = d // 2
    zh = half // 2
    fc = f // _C

    def body(x_ref, dy_ref, out_ref, dyv, xt, ysend, yrecv, zsend, zrecv,
             zredf, zof, dy_sem, out_sem,
             ysend_sem, yrecv_sem, zsend_sem, zrecv_sem):
        my_x = lax.axis_index("x")
        my_y = lax.axis_index("y")
        my_z = lax.axis_index("z")
        ypartner = (my_x, 1 - my_y, my_z)
        zpartner = (my_x, my_y, 1 - my_z)

        dy_cp = pltpu.make_async_copy(dy_ref, dyv, dy_sem)
        dy_cp.start()

        barrier_sem = pltpu.get_barrier_semaphore()
        for nbr in (ypartner, zpartner):
            pl.semaphore_signal(
                barrier_sem, inc=1, device_id=nbr,
                device_id_type=pl.DeviceIdType.MESH,
            )
        pl.semaphore_wait(barrier_sem, 2)

        xt[0] = x_ref[:, pl.ds((1 - my_y) * half + my_z * zh, zh)].T
        xt[1] = x_ref[:, pl.ds(my_y * half + my_z * zh, zh)].T
        xs = xt[0]
        xo = xt[1]

        dy_cp.wait()

        y_rdmas = []
        for i in range(_C):
            ps = lax.dot_general(
                xs, dyv[:, i * fc:(i + 1) * fc], _DIMS,
                preferred_element_type=jnp.float32,
            )
            ysend[i] = ps.astype(jnp.bfloat16)
            r = pltpu.make_async_remote_copy(
                src_ref=ysend.at[i], dst_ref=yrecv.at[i],
                send_sem=ysend_sem.at[i], recv_sem=yrecv_sem.at[i],
                device_id=ypartner, device_id_type=pl.DeviceIdType.MESH,
            )
            r.start()
            y_rdmas.append(r)

        z_rdmas = []
        out_cps = []
        for i in range(_C):
            own = lax.dot_general(
                xo, dyv[:, i * fc:(i + 1) * fc], _DIMS,
                preferred_element_type=jnp.float32,
            )
            y_rdmas[i].wait_recv()
            red = own + yrecv[i].astype(jnp.float32)
            zredf[i] = red
            zsend[i] = red.astype(jnp.bfloat16)
            r = pltpu.make_async_remote_copy(
                src_ref=zsend.at[i], dst_ref=zrecv.at[i],
                send_sem=zsend_sem.at[i], recv_sem=zrecv_sem.at[i],
                device_id=zpartner, device_id_type=pl.DeviceIdType.MESH,
            )
            r.start()
            z_rdmas.append(r)
            cp = pltpu.make_async_copy(
                zredf.at[i],
                out_ref.at[pl.ds(my_z * zh, zh), pl.ds(i * fc, fc)],
                out_sem.at[i],
            )
            cp.start()
            out_cps.append(cp)

        for i in range(_C):
            z_rdmas[i].wait_recv()
            zof[i] = zrecv[i].astype(jnp.float32)
            cp = pltpu.make_async_copy(
                zof.at[i],
                out_ref.at[pl.ds((1 - my_z) * zh, zh), pl.ds(i * fc, fc)],
                out_sem.at[_C + i],
            )
            cp.start()
            out_cps.append(cp)

        for cp in out_cps:
            cp.wait()
        for i in range(_C):
            y_rdmas[i].wait_send()
            z_rdmas[i].wait_send()

    return pl.pallas_call(
        body,
        out_shape=jax.ShapeDtypeStruct((half, f), jnp.float32),
        in_specs=[
            pl.BlockSpec(memory_space=pltpu.VMEM),
            pl.BlockSpec(memory_space=pltpu.MemorySpace.HBM),
        ],
        out_specs=pl.BlockSpec(memory_space=pltpu.MemorySpace.HBM),
        scratch_shapes=[
            pltpu.VMEM((m, f), jnp.float32),
            pltpu.VMEM((2, zh, m), jnp.float32),
            pltpu.VMEM((_C, zh, fc), jnp.bfloat16),
            pltpu.VMEM((_C, zh, fc), jnp.bfloat16),
            pltpu.VMEM((_C, zh, fc), jnp.bfloat16),
            pltpu.VMEM((_C, zh, fc), jnp.bfloat16),
            pltpu.VMEM((_C, zh, fc), jnp.float32),
            pltpu.VMEM((_C, zh, fc), jnp.float32),
            pltpu.SemaphoreType.DMA,
            pltpu.SemaphoreType.DMA((2 * _C,)),
            pltpu.SemaphoreType.DMA((_C,)),
            pltpu.SemaphoreType.DMA((_C,)),
            pltpu.SemaphoreType.DMA((_C,)),
            pltpu.SemaphoreType.DMA((_C,)),
        ],
        compiler_params=pltpu.CompilerParams(collective_id=0),
    )(x, dy)
